# baseline (speedup 1.0000x reference)
import functools

import jax
import jax.numpy as jnp
from jax import lax
from jax.experimental import pallas as pl
from jax.experimental.pallas import tpu as pltpu
from jax.experimental.pallas import tpu_sc as plsc

N = 10000
E = 320000
D = 128
NCLS = 10
L = 4

NUM_SC = 2
NUM_TILES = 16
CHUNK = 64
R = 160
NB = 4
NPHASE = 4
RP = R // NPHASE
E_PAD = NUM_SC * NUM_TILES * R * CHUNK
N_PAD = 10112
ROWS_PER_TILE = N_PAD // NUM_TILES
DUMMY_DST = 10008
DEG_W = 16
DEG_CHUNK = 128
DEG_R = E_PAD // (NUM_SC * NUM_TILES * DEG_CHUNK)

_mesh = plsc.VectorSubcoreMesh(core_axis_name="c", subcore_axis_name="s")


@functools.partial(
    pl.kernel,
    out_type=jax.ShapeDtypeStruct((NUM_SC, N_PAD, D), jnp.float32),
    mesh=_mesh,
    scratch_types=[
        pltpu.VMEM((RP, CHUNK), jnp.int32),
        pltpu.VMEM((RP, CHUNK), jnp.int32),
        pltpu.VMEM((NB, CHUNK, D), jnp.float32),
        pltpu.VMEM_SHARED((N_PAD, D), jnp.float32),
        [pltpu.SemaphoreType.DMA] * NB,
        [pltpu.SemaphoreType.DMA] * NB,
    ],
)
def _mp(src_hbm, dst_hbm, h_hbm, zeros_hbm, out_hbm,
        src_v, dst_v, rows_v, acc_sh, gsems, ssems):
    c = lax.axis_index("c")
    s = lax.axis_index("s")
    wid = c * NUM_TILES + s
    r0 = s * ROWS_PER_TILE
    pltpu.sync_copy(zeros_hbm.at[pl.ds(r0, ROWS_PER_TILE)],
                    acc_sh.at[pl.ds(r0, ROWS_PER_TILE)])
    plsc.subcore_barrier()

    for phase in range(NPHASE):
        base = wid * R + phase * RP
        pltpu.sync_copy(src_hbm.at[pl.ds(base, RP)], src_v)
        pltpu.sync_copy(dst_hbm.at[pl.ds(base, RP)], dst_v)
        for b in range(NB):
            pltpu.async_copy(h_hbm.at[src_v.at[b]], rows_v.at[b], gsems[b])

        @pl.loop(0, RP // NB)
        def _(i):
            j = i * NB
            for b in range(NB):
                pltpu.make_async_copy(h_hbm.at[src_v.at[j + b]], rows_v.at[b],
                                      gsems[b]).wait()
            for b in range(NB):
                jn = lax.rem(j + NB + b, RP)
                pltpu.async_copy(h_hbm.at[src_v.at[jn]], rows_v.at[b],
                                 gsems[b])

        for b in range(NB):
            pltpu.make_async_copy(h_hbm.at[src_v.at[b]], rows_v.at[b],
                                  gsems[b]).wait()

    plsc.subcore_barrier()
    pltpu.sync_copy(acc_sh.at[pl.ds(r0, ROWS_PER_TILE)],
                    out_hbm.at[c].at[pl.ds(r0, ROWS_PER_TILE)])


def _dense_body(relu, residual,
                parts_ref, degp_ref, hin_ref, w_ref, b_ref, gamma_ref,
                beta_ref, snorm_ref, out_ref):
    agg = parts_ref[0, :N, :] + parts_ref[1, :N, :]
    deg = degp_ref[0, :N, 0:1] + degp_ref[1, :N, 0:1]
    x = agg * (1.0 / jnp.maximum(deg, 1.0))
    y = jnp.dot(x, w_ref[...], preferred_element_type=jnp.float32)
    y = (y + b_ref[...]) * snorm_ref[...]
    mean = jnp.mean(y, axis=0, keepdims=True)
    var = jnp.mean((y - mean) * (y - mean), axis=0, keepdims=True)
    y = (y - mean) * lax.rsqrt(var + 1e-5) * gamma_ref[...] + beta_ref[...]
    if relu:
        y = jnp.maximum(y, 0.0)
    if residual:
        y = y + hin_ref[...]
    out_ref[...] = y


def _dense(parts, degp, h_in, w, b, gamma, beta, snorm, relu, residual):
    return pl.pallas_call(
        functools.partial(_dense_body, relu, residual),
        out_shape=jax.ShapeDtypeStruct((N, D), jnp.float32),
    )(parts, degp, h_in, w, b, gamma, beta, snorm)


def kernel(h, edge_index, e, snorm_n, snorm_e, W_h, b_h, gamma_h, beta_h,
           W_out, b_out, gamma_out, beta_out):
    del e, snorm_e
    src = edge_index[0]
    dst = edge_index[1]
    pad = E_PAD - E
    src_p = jnp.concatenate([src, jnp.zeros((pad,), jnp.int32)])
    dst_p = jnp.concatenate([dst, jnp.full((pad,), DUMMY_DST, jnp.int32)])
    src2d = src_p.reshape(E_PAD // CHUNK, CHUNK)
    dst2d = dst_p.reshape(E_PAD // CHUNK, CHUNK)
    zeros = jnp.zeros((N_PAD, D), jnp.float32)

    degp = _mp(src2d, dst2d, jnp.ones((N, D), jnp.float32), zeros)

    b2 = b_h.reshape(L, 1, D)
    g2 = gamma_h.reshape(L, 1, D)
    be2 = beta_h.reshape(L, 1, D)
    w_out_p = jnp.zeros((D, D), jnp.float32).at[:, :NCLS].set(W_out)
    b_out_p = jnp.zeros((1, D), jnp.float32).at[0, :NCLS].set(b_out)
    g_out_p = jnp.ones((1, D), jnp.float32).at[0, :NCLS].set(gamma_out)
    be_out_p = jnp.zeros((1, D), jnp.float32).at[0, :NCLS].set(beta_out)

    hc = h
    for i in range(L):
        parts = _mp(src2d, dst2d, hc, zeros)
        hc = _dense(parts, degp, hc, W_h[i], b2[i], g2[i], be2[i], snorm_n,
                    relu=True, residual=True)
    parts = _mp(src2d, dst2d, hc, zeros)
    out = _dense(parts, degp, hc, w_out_p, b_out_p, g_out_p, be_out_p,
                 snorm_n, relu=False, residual=False)
    return out[:, :NCLS]


# speedup vs baseline: 1.1430x; 1.1430x over previous
import functools

import jax
import jax.numpy as jnp
from jax import lax
from jax.experimental import pallas as pl
from jax.experimental.pallas import tpu as pltpu
from jax.experimental.pallas import tpu_sc as plsc

N = 10000
E = 320000
D = 128
NCLS = 10
L = 4

NUM_SC = 2
NUM_TILES = 16
CHUNK = 64
R = 160
NB = 4
NPHASE = 4
RP = R // NPHASE
E_PAD = NUM_SC * NUM_TILES * R * CHUNK
N_PAD = 10112
ROWS_PER_TILE = N_PAD // NUM_TILES
DUMMY_DST = 10008
DEG_W = 16
DEG_CHUNK = 128
DEG_R = E_PAD // (NUM_SC * NUM_TILES * DEG_CHUNK)

_mesh = plsc.VectorSubcoreMesh(core_axis_name="c", subcore_axis_name="s")


@functools.partial(
    pl.kernel,
    out_type=jax.ShapeDtypeStruct((NUM_SC, N_PAD, D), jnp.float32),
    mesh=_mesh,
    scratch_types=[
        pltpu.VMEM((RP, CHUNK), jnp.int32),
        pltpu.VMEM((RP, CHUNK), jnp.int32),
        pltpu.VMEM((NB, CHUNK, D), jnp.float32),
        pltpu.VMEM_SHARED((N_PAD, D), jnp.float32),
        [pltpu.SemaphoreType.DMA] * NB,
        [pltpu.SemaphoreType.DMA] * NB,
    ],
)
def _mp(src_hbm, dst_hbm, h_hbm, zeros_hbm, out_hbm,
        src_v, dst_v, rows_v, acc_sh, gsems, ssems):
    c = lax.axis_index("c")
    s = lax.axis_index("s")
    wid = c * NUM_TILES + s
    r0 = s * ROWS_PER_TILE
    pltpu.sync_copy(zeros_hbm.at[pl.ds(r0, ROWS_PER_TILE)],
                    acc_sh.at[pl.ds(r0, ROWS_PER_TILE)])
    plsc.subcore_barrier()

    for phase in range(NPHASE):
        base = wid * R + phase * RP
        pltpu.sync_copy(src_hbm.at[pl.ds(base, RP)], src_v)
        pltpu.sync_copy(dst_hbm.at[pl.ds(base, RP)], dst_v)
        for b in range(NB):
            pltpu.async_copy(h_hbm.at[src_v.at[b]], rows_v.at[b], gsems[b])

        @pl.loop(0, RP // NB)
        def _(i):
            j = i * NB
            for b in range(NB):
                pltpu.make_async_copy(h_hbm.at[src_v.at[j + b]], rows_v.at[b],
                                      gsems[b]).wait()
                pltpu.async_copy(rows_v.at[b], acc_sh.at[dst_v.at[j + b]],
                                 ssems[b], add=True)
            for b in range(NB):
                pltpu.make_async_copy(rows_v.at[b], acc_sh.at[dst_v.at[j + b]],
                                      ssems[b]).wait()
                jn = lax.rem(j + NB + b, RP)
                pltpu.async_copy(h_hbm.at[src_v.at[jn]], rows_v.at[b],
                                 gsems[b])

        for b in range(NB):
            pltpu.make_async_copy(h_hbm.at[src_v.at[b]], rows_v.at[b],
                                  gsems[b]).wait()

    plsc.subcore_barrier()
    pltpu.sync_copy(acc_sh.at[pl.ds(r0, ROWS_PER_TILE)],
                    out_hbm.at[c].at[pl.ds(r0, ROWS_PER_TILE)])


@functools.partial(
    pl.kernel,
    out_type=jax.ShapeDtypeStruct((NUM_SC, N_PAD, D), jnp.float32),
    mesh=_mesh,
    scratch_types=[
        pltpu.VMEM((DEG_R, DEG_CHUNK), jnp.int32),
        pltpu.VMEM((DEG_CHUNK, D), jnp.float32),
        pltpu.VMEM_SHARED((N_PAD, D), jnp.float32),
        [pltpu.SemaphoreType.DMA] * NB,
    ],
)
def _deg(dst_hbm, zeros_hbm, ones_hbm, out_hbm, dst_v, ones_v, deg_sh, ssems):
    c = lax.axis_index("c")
    s = lax.axis_index("s")
    wid = c * NUM_TILES + s
    r0 = s * ROWS_PER_TILE
    pltpu.sync_copy(zeros_hbm.at[pl.ds(r0, ROWS_PER_TILE)],
                    deg_sh.at[pl.ds(r0, ROWS_PER_TILE)])
    pltpu.sync_copy(ones_hbm, ones_v)
    pltpu.sync_copy(dst_hbm.at[pl.ds(wid * DEG_R, DEG_R)], dst_v)
    plsc.subcore_barrier()

    for b in range(NB):
        pltpu.async_copy(ones_v, deg_sh.at[dst_v.at[b]], ssems[b], add=True)

    @pl.loop(0, DEG_R // NB - 1)
    def _(i):
        j = i * NB
        for b in range(NB):
            pltpu.make_async_copy(ones_v, deg_sh.at[dst_v.at[j + b]],
                                  ssems[b]).wait()
            pltpu.async_copy(ones_v, deg_sh.at[dst_v.at[j + NB + b]],
                             ssems[b], add=True)

    for b in range(NB):
        pltpu.make_async_copy(ones_v, deg_sh.at[dst_v.at[DEG_R - NB + b]],
                              ssems[b]).wait()

    plsc.subcore_barrier()
    pltpu.sync_copy(deg_sh.at[pl.ds(r0, ROWS_PER_TILE)],
                    out_hbm.at[c].at[pl.ds(r0, ROWS_PER_TILE)])


def _dense_body(relu, residual,
                parts_ref, degp_ref, hin_ref, w_ref, b_ref, gamma_ref,
                beta_ref, snorm_ref, out_ref):
    agg = parts_ref[0, :N, :] + parts_ref[1, :N, :]
    deg = degp_ref[0, :N, 0:1] + degp_ref[1, :N, 0:1]
    x = agg * (1.0 / jnp.maximum(deg, 1.0))
    y = jnp.dot(x, w_ref[...], preferred_element_type=jnp.float32)
    y = (y + b_ref[...]) * snorm_ref[...]
    mean = jnp.mean(y, axis=0, keepdims=True)
    var = jnp.mean((y - mean) * (y - mean), axis=0, keepdims=True)
    y = (y - mean) * lax.rsqrt(var + 1e-5) * gamma_ref[...] + beta_ref[...]
    if relu:
        y = jnp.maximum(y, 0.0)
    if residual:
        y = y + hin_ref[...]
    out_ref[...] = y


def _dense(parts, degp, h_in, w, b, gamma, beta, snorm, relu, residual):
    return pl.pallas_call(
        functools.partial(_dense_body, relu, residual),
        out_shape=jax.ShapeDtypeStruct((N, D), jnp.float32),
    )(parts, degp, h_in, w, b, gamma, beta, snorm)


def kernel(h, edge_index, e, snorm_n, snorm_e, W_h, b_h, gamma_h, beta_h,
           W_out, b_out, gamma_out, beta_out):
    del e, snorm_e
    src = edge_index[0]
    dst = edge_index[1]
    pad = E_PAD - E
    src_p = jnp.concatenate([src, jnp.zeros((pad,), jnp.int32)])
    dst_p = jnp.concatenate([dst, jnp.full((pad,), DUMMY_DST, jnp.int32)])
    src2d = src_p.reshape(E_PAD // CHUNK, CHUNK)
    dst2d = dst_p.reshape(E_PAD // CHUNK, CHUNK)
    dst2d_deg = dst_p.reshape(E_PAD // DEG_CHUNK, DEG_CHUNK)
    zeros = jnp.zeros((N_PAD, D), jnp.float32)
    ones_rows = jnp.ones((DEG_CHUNK, D), jnp.float32)

    degp = _deg(dst2d_deg, zeros, ones_rows)

    b2 = b_h.reshape(L, 1, D)
    g2 = gamma_h.reshape(L, 1, D)
    be2 = beta_h.reshape(L, 1, D)
    w_out_p = jnp.zeros((D, D), jnp.float32).at[:, :NCLS].set(W_out)
    b_out_p = jnp.zeros((1, D), jnp.float32).at[0, :NCLS].set(b_out)
    g_out_p = jnp.ones((1, D), jnp.float32).at[0, :NCLS].set(gamma_out)
    be_out_p = jnp.zeros((1, D), jnp.float32).at[0, :NCLS].set(beta_out)

    hc = h
    for i in range(L):
        parts = _mp(src2d, dst2d, hc, zeros)
        hc = _dense(parts, degp, hc, W_h[i], b2[i], g2[i], be2[i], snorm_n,
                    relu=True, residual=True)
    parts = _mp(src2d, dst2d, hc, zeros)
    out = _dense(parts, degp, hc, w_out_p, b_out_p, g_out_p, be_out_p,
                 snorm_n, relu=False, residual=False)
    return out[:, :NCLS]
